# baseline (speedup 1.0000x reference)
import functools

import jax
import jax.numpy as jnp
from jax.experimental import pallas as pl
from jax.experimental.pallas import tpu as pltpu

_CHUNK_ROWS = 1024
_N_SLOTS = 4


def _gating_body(c_rows, n_slots, x_hbm, sim_ref, gates_ref, temp_ref,
                 em_ref, mask_ref, logits_ref, buf_ref, simn_ref, sem):
    i = pl.program_id(0)
    n = pl.num_programs(0)
    la = n_slots - 1
    slot = jax.lax.rem(i, n_slots)
    nslot = jax.lax.rem(i + la, n_slots)

    @pl.when(i == 0)
    def _():
        pltpu.make_async_copy(
            x_hbm.at[pl.ds(0, c_rows), :], buf_ref.at[0], sem.at[0]
        ).start()
        sim = sim_ref[...]
        col_n = jnp.sqrt(jnp.sum(sim * sim, axis=0, keepdims=True))
        simn_ref[...] = (sim / jnp.maximum(col_n, 1e-12)).astype(jnp.bfloat16)
        for k in range(1, la):
            pltpu.make_async_copy(
                x_hbm.at[pl.ds(k * c_rows, c_rows), :],
                buf_ref.at[k],
                sem.at[k],
            ).start()

    if la >= 1:
        @pl.when(i + la < n)
        def _():
            pltpu.make_async_copy(
                x_hbm.at[pl.ds((i + la) * c_rows, c_rows), :],
                buf_ref.at[nslot],
                sem.at[nslot],
            ).start()

    pltpu.make_async_copy(
        x_hbm.at[pl.ds(i * c_rows, c_rows), :], buf_ref.at[slot], sem.at[slot]
    ).wait()

    x = buf_ref[slot]

    row_n = jnp.sqrt(jnp.sum(x * x, axis=1, keepdims=True))
    x_n = (x / jnp.maximum(row_n, 1e-12)).astype(jnp.bfloat16)

    logits = jnp.dot(x_n, simn_ref[...], preferred_element_type=jnp.float32)
    logits = logits * em_ref[...]

    scaled_gates = gates_ref[...] * jax.nn.sigmoid(temp_ref[...])
    act = (logits > scaled_gates).astype(jnp.float32)
    inactive = jnp.sum(act, axis=1, keepdims=True) == 0.0

    n_e = logits.shape[1]
    idx = jax.lax.broadcasted_iota(jnp.int32, logits.shape, 1).astype(jnp.float32)
    neg_big = jnp.float32(jnp.finfo(jnp.float32).min)
    m1 = jnp.max(logits, axis=1, keepdims=True)
    i1 = jnp.min(jnp.where(logits == m1, idx, jnp.float32(n_e)),
                 axis=1, keepdims=True)
    oh1 = idx == i1
    rest = jnp.where(oh1, neg_big, logits)
    m2 = jnp.max(rest, axis=1, keepdims=True)
    i2 = jnp.min(jnp.where(rest == m2, idx, jnp.float32(n_e)),
                 axis=1, keepdims=True)
    fallback = jnp.logical_or(oh1, idx == i2).astype(jnp.float32)

    mask_ref[...] = jnp.where(inactive, fallback, act)
    logits_ref[...] = logits


def kernel(hidden_states, sim_matrix, gates, temperature, experts_mask):
    b, t, h = hidden_states.shape
    n_e = sim_matrix.shape[1]
    rows = b * t
    flat = hidden_states.reshape(rows, h)

    c_rows = min(_CHUNK_ROWS, rows)
    n_chunks = rows // c_rows
    n_slots = min(_N_SLOTS, n_chunks)

    out_shapes = (
        jax.ShapeDtypeStruct((rows, n_e), jnp.float32),
        jax.ShapeDtypeStruct((rows, n_e), jnp.float32),
    )

    mask, logits = pl.pallas_call(
        functools.partial(_gating_body, c_rows, n_slots),
        grid=(n_chunks,),
        in_specs=[
            pl.BlockSpec(memory_space=pl.ANY),
            pl.BlockSpec((h, n_e), lambda i: (0, 0)),
            pl.BlockSpec((1, n_e), lambda i: (0, 0)),
            pl.BlockSpec((1, 1), lambda i: (0, 0)),
            pl.BlockSpec((1, n_e), lambda i: (0, 0)),
        ],
        out_specs=(
            pl.BlockSpec((c_rows, n_e), lambda i: (i, 0)),
            pl.BlockSpec((c_rows, n_e), lambda i: (i, 0)),
        ),
        out_shape=out_shapes,
        scratch_shapes=[
            pltpu.VMEM((n_slots, c_rows, h), jnp.float32),
            pltpu.VMEM((h, n_e), jnp.bfloat16),
            pltpu.SemaphoreType.DMA((n_slots,)),
        ],
        compiler_params=pltpu.CompilerParams(
            dimension_semantics=("arbitrary",),
        ),
    )(
        flat,
        sim_matrix,
        gates.reshape(1, n_e),
        temperature.reshape(1, 1),
        experts_mask.reshape(1, n_e),
    )
    return mask, logits


# speedup vs baseline: 1.0123x; 1.0123x over previous
import functools

import jax
import jax.numpy as jnp
from jax.experimental import pallas as pl
from jax.experimental.pallas import tpu as pltpu

_CHUNK_ROWS = 512
_N_SLOTS = 4


def _gating_body(c_rows, n_slots, x_hbm, sim_ref, gates_ref, temp_ref,
                 em_ref, mask_ref, logits_ref, buf_ref, simn_ref, sem):
    i = pl.program_id(0)
    n = pl.num_programs(0)
    la = n_slots - 1
    slot = jax.lax.rem(i, n_slots)
    nslot = jax.lax.rem(i + la, n_slots)

    @pl.when(i == 0)
    def _():
        pltpu.make_async_copy(
            x_hbm.at[pl.ds(0, c_rows), :], buf_ref.at[0], sem.at[0]
        ).start()
        sim = sim_ref[...]
        col_n = jnp.sqrt(jnp.sum(sim * sim, axis=0, keepdims=True))
        simn_ref[...] = (sim / jnp.maximum(col_n, 1e-12)).astype(jnp.bfloat16)
        for k in range(1, la):
            pltpu.make_async_copy(
                x_hbm.at[pl.ds(k * c_rows, c_rows), :],
                buf_ref.at[k],
                sem.at[k],
            ).start()

    if la >= 1:
        @pl.when(i + la < n)
        def _():
            pltpu.make_async_copy(
                x_hbm.at[pl.ds((i + la) * c_rows, c_rows), :],
                buf_ref.at[nslot],
                sem.at[nslot],
            ).start()

    pltpu.make_async_copy(
        x_hbm.at[pl.ds(i * c_rows, c_rows), :], buf_ref.at[slot], sem.at[slot]
    ).wait()

    x = buf_ref[slot]

    row_n = jnp.sqrt(jnp.sum(x * x, axis=1, keepdims=True))
    x_n = (x / jnp.maximum(row_n, 1e-12)).astype(jnp.bfloat16)

    logits = jnp.dot(x_n, simn_ref[...], preferred_element_type=jnp.float32)
    logits = logits * em_ref[...]

    scaled_gates = gates_ref[...] * jax.nn.sigmoid(temp_ref[...])
    act = (logits > scaled_gates).astype(jnp.float32)
    inactive = jnp.sum(act, axis=1, keepdims=True) == 0.0

    n_e = logits.shape[1]
    idx = jax.lax.broadcasted_iota(jnp.int32, logits.shape, 1).astype(jnp.float32)
    neg_big = jnp.float32(jnp.finfo(jnp.float32).min)
    m1 = jnp.max(logits, axis=1, keepdims=True)
    i1 = jnp.min(jnp.where(logits == m1, idx, jnp.float32(n_e)),
                 axis=1, keepdims=True)
    oh1 = idx == i1
    rest = jnp.where(oh1, neg_big, logits)
    m2 = jnp.max(rest, axis=1, keepdims=True)
    i2 = jnp.min(jnp.where(rest == m2, idx, jnp.float32(n_e)),
                 axis=1, keepdims=True)
    fallback = jnp.logical_or(oh1, idx == i2).astype(jnp.float32)

    mask_ref[...] = jnp.where(inactive, fallback, act)
    logits_ref[...] = logits


def kernel(hidden_states, sim_matrix, gates, temperature, experts_mask):
    b, t, h = hidden_states.shape
    n_e = sim_matrix.shape[1]
    rows = b * t
    flat = hidden_states.reshape(rows, h)

    c_rows = min(_CHUNK_ROWS, rows)
    n_chunks = rows // c_rows
    n_slots = min(_N_SLOTS, n_chunks)

    out_shapes = (
        jax.ShapeDtypeStruct((rows, n_e), jnp.float32),
        jax.ShapeDtypeStruct((rows, n_e), jnp.float32),
    )

    mask, logits = pl.pallas_call(
        functools.partial(_gating_body, c_rows, n_slots),
        grid=(n_chunks,),
        in_specs=[
            pl.BlockSpec(memory_space=pl.ANY),
            pl.BlockSpec((h, n_e), lambda i: (0, 0)),
            pl.BlockSpec((1, n_e), lambda i: (0, 0)),
            pl.BlockSpec((1, 1), lambda i: (0, 0)),
            pl.BlockSpec((1, n_e), lambda i: (0, 0)),
        ],
        out_specs=(
            pl.BlockSpec((c_rows, n_e), lambda i: (i, 0)),
            pl.BlockSpec((c_rows, n_e), lambda i: (i, 0)),
        ),
        out_shape=out_shapes,
        scratch_shapes=[
            pltpu.VMEM((n_slots, c_rows, h), jnp.float32),
            pltpu.VMEM((h, n_e), jnp.bfloat16),
            pltpu.SemaphoreType.DMA((n_slots,)),
        ],
        compiler_params=pltpu.CompilerParams(
            dimension_semantics=("arbitrary",),
        ),
    )(
        flat,
        sim_matrix,
        gates.reshape(1, n_e),
        temperature.reshape(1, 1),
        experts_mask.reshape(1, n_e),
    )
    return mask, logits
